# baseline (speedup 1.0000x reference)
import functools

import jax
import jax.numpy as jnp
from jax import lax
from jax.experimental import pallas as pl
from jax.experimental.pallas import tpu as pltpu
from jax.experimental.pallas import tpu_sc as plsc

_DIM = 64
_VOCAB = 1000000
_NFULL = _VOCAB // 128
_TAIL = _VOCAB - _NFULL * 128


_info = plsc.get_sparse_core_info()
_NC, _NS = _info.num_cores, _info.num_subcores
_NW = _NC * _NS


def _transpose_cols(slab, stage, n):
    def row(v, _):
        for c in range(4):
            rows16 = lax.iota(jnp.int32, 16) + (c * 16)
            cols16 = lax.iota(jnp.int32, 16) * 0 + v
            vals = plsc.load_gather(slab, [rows16, cols16])
            stage[v, pl.ds(c * 16, 16)] = vals
        return ()
    lax.fori_loop(0, n, row, (), unroll=False)


def _tr_body(tabT_hbm, q_hbm, slab_v, stage_v, tslab_v, tstage_v):
    wid = lax.axis_index("s") * _NC + lax.axis_index("c")

    def blk(k, _):
        b = wid + k * _NW

        @pl.when(b < _NFULL)
        def _():
            v0 = b * 128
            pltpu.sync_copy(tabT_hbm.at[:, pl.ds(v0, 128)], slab_v)
            _transpose_cols(slab_v, stage_v, 128)
            pltpu.sync_copy(stage_v, q_hbm.at[pl.ds(v0, 128)])
        return ()

    nblk = _NFULL // _NW + 1
    lax.fori_loop(0, nblk, blk, (), unroll=False)

    @pl.when(wid == 0)
    def _():
        v0 = _NFULL * 128
        pltpu.sync_copy(tabT_hbm.at[:, pl.ds(v0, _TAIL)], tslab_v)
        _transpose_cols(tslab_v, tstage_v, _TAIL)
        pltpu.sync_copy(tstage_v, q_hbm.at[pl.ds(v0, _TAIL)])


def _sc_transpose(tabT):
    mesh = plsc.VectorSubcoreMesh(core_axis_name="c", subcore_axis_name="s")
    return pl.kernel(
        _tr_body,
        out_type=jax.ShapeDtypeStruct((_VOCAB, 128), jnp.float32),
        mesh=mesh,
        scratch_types=[
            pltpu.VMEM((_DIM, 128), jnp.float32),
            pltpu.VMEM((128, 128), jnp.float32),
            pltpu.VMEM((_DIM, _TAIL), jnp.float32),
            pltpu.VMEM((_TAIL, 128), jnp.float32),
        ],
        compiler_params=pltpu.CompilerParams(needs_layout_passes=False),
    )(tabT)


_SUB = 128
_CHUNK = 512


def _gather_body(idx_hbm, q_hbm, out_hbm, idx_v, rows_v, gsem, bpw):
    wid = lax.axis_index("s") * _NC + lax.axis_index("c")
    base = wid * bpw
    pltpu.sync_copy(idx_hbm.at[pl.ds(base, bpw)], idx_v)

    def chunk(ci, _):
        off = ci * _CHUNK
        handles = []
        for j in range(_CHUNK // _SUB):
            handles.append(pltpu.async_copy(
                q_hbm.at[idx_v.at[pl.ds(off + j * _SUB, _SUB)]],
                rows_v.at[pl.ds(j * _SUB, _SUB)],
                gsem,
            ))
        for h in handles:
            h.wait()
        pltpu.sync_copy(rows_v, out_hbm.at[pl.ds(base + off, _CHUNK)])
        return ()

    lax.fori_loop(0, bpw // _CHUNK, chunk, (), unroll=False)


def _sc_gather(idx_flat, q):
    n = idx_flat.shape[0]
    assert n % (_NW * _CHUNK) == 0
    bpw = n // _NW
    mesh = plsc.VectorSubcoreMesh(core_axis_name="c", subcore_axis_name="s")
    body = functools.partial(_gather_body, bpw=bpw)
    return pl.kernel(
        body,
        out_type=jax.ShapeDtypeStruct((n, 128), jnp.float32),
        mesh=mesh,
        scratch_types=[
            pltpu.VMEM((bpw,), jnp.int32),
            pltpu.VMEM((_CHUNK, 128), jnp.float32),
            pltpu.SemaphoreType.DMA,
        ],
    )(idx_flat, q)


_ROWS = 2048


def _proj_body(emb_ref, w_ref, b_ref, out_ref):
    e = emb_ref[0][:, 0:_DIM]
    acc = lax.dot_general(
        w_ref[...], e, (((0,), (1,)), ((), ())),
        preferred_element_type=jnp.float32,
    )
    acc = acc + b_ref[...]
    out_ref[0] = acc * jax.nn.sigmoid(acc)


def _tc_project(emb3, W, bcol):
    F, B = emb3.shape[0], emb3.shape[1]
    return pl.pallas_call(
        _proj_body,
        grid=(F, B // _ROWS),
        in_specs=[
            pl.BlockSpec((1, _ROWS, 128), lambda f, i: (f, i, 0)),
            pl.BlockSpec((_DIM, _DIM), lambda f, i: (0, 0)),
            pl.BlockSpec((_DIM, 1), lambda f, i: (0, 0)),
        ],
        out_specs=pl.BlockSpec((1, _DIM, _ROWS), lambda f, i: (f, 0, i)),
        out_shape=jax.ShapeDtypeStruct((F, _DIM, B), jnp.float32),
    )(emb3, W, bcol)


def kernel(x, table, W, b):
    B, F = x.shape
    idx_flat = x.T.reshape(-1)
    tabT = table.T
    q = _sc_transpose(tabT)
    emb = _sc_gather(idx_flat, q)
    emb3 = emb.reshape(F, B, 128)
    out3 = _tc_project(emb3, W, b.reshape(_DIM, 1))
    return out3.transpose(2, 0, 1)


# speedup vs baseline: 1.2279x; 1.2279x over previous
import functools

import jax
import jax.numpy as jnp
from jax import lax
from jax.experimental import pallas as pl
from jax.experimental.pallas import tpu as pltpu
from jax.experimental.pallas import tpu_sc as plsc

_DIM = 64
_VOCAB = 1000000
_NFULL = _VOCAB // 128
_TAIL = _VOCAB - _NFULL * 128


_info = plsc.get_sparse_core_info()
_NC, _NS = _info.num_cores, _info.num_subcores
_NW = _NC * _NS


_PITCH = 128


def _transpose_cols(slab, stage, n):
    def row(v, _):
        for c in range(4):
            rows16 = lax.iota(jnp.int32, 16) + (c * 16)
            cols16 = lax.iota(jnp.int32, 16) * 0 + v
            vals = plsc.load_gather(slab, [rows16, cols16])
            stage[v, pl.ds(c * 16, 16)] = vals
        return ()
    lax.fori_loop(0, n, row, (), unroll=8)


def _tr_body(tabT_hbm, q_hbm, slabs, stages, isem, osem, tail_hbm):
    wid = lax.axis_index("s") * _NC + lax.axis_index("c")

    def in_copy(b, d):
        return pltpu.make_async_copy(
            tabT_hbm.at[:, pl.ds(b * 128, 128)], slabs[d], isem)

    def out_copy(b, d):
        return pltpu.make_async_copy(
            stages[d], q_hbm.at[pl.ds(b * 128, 128)], osem)

    in_copy(wid, 0).start()
    in_copy(wid + _NW, 1).start()

    nblk = _NFULL // _NW + 1

    def blk2(k0, _):
        for d in range(2):
            k = k0 + d
            b = wid + k * _NW

            @pl.when(b < _NFULL)
            def _():
                in_copy(b, d).wait()

                @pl.when(k >= 2)
                def _():
                    out_copy(b - 2 * _NW, d).wait()
                _transpose_cols(slabs[d], stages[d], 128)
                out_copy(b, d).start()

                @pl.when(b + 2 * _NW < _NFULL)
                def _():
                    in_copy(b + 2 * _NW, d).start()
        return ()

    lax.fori_loop(0, nblk // 2 + 1, lambda k, c: blk2(2 * k, c), (),
                  unroll=False)

    nv = _NFULL // _NW + jnp.where(wid < _NFULL % _NW, 1, 0)
    out_copy(wid + (nv - 2) * _NW, 0).wait()
    out_copy(wid + (nv - 1) * _NW, 1).wait()

    @pl.when(wid == 0)
    def _():
        pltpu.sync_copy(tail_hbm, stages[0].at[pl.ds(0, _TAIL)])
        pltpu.sync_copy(stages[0].at[pl.ds(0, _TAIL)],
                        q_hbm.at[pl.ds(_NFULL * 128, _TAIL)])


def _tr_body_wrap(tabT_hbm, tail_hbm, q_hbm, slab_a, slab_b, stage_a,
                  stage_b, isem, osem):
    _tr_body(tabT_hbm, q_hbm, (slab_a, slab_b), (stage_a, stage_b),
             isem, osem, tail_hbm)


def _sc_transpose(tabT, tail_pad):
    mesh = plsc.VectorSubcoreMesh(core_axis_name="c", subcore_axis_name="s")
    return pl.kernel(
        _tr_body_wrap,
        out_type=jax.ShapeDtypeStruct((_VOCAB, 128), jnp.float32),
        mesh=mesh,
        scratch_types=[
            pltpu.VMEM((_DIM, _PITCH), jnp.float32),
            pltpu.VMEM((_DIM, _PITCH), jnp.float32),
            pltpu.VMEM((128, 128), jnp.float32),
            pltpu.VMEM((128, 128), jnp.float32),
            pltpu.SemaphoreType.DMA,
            pltpu.SemaphoreType.DMA,
        ],
        compiler_params=pltpu.CompilerParams(needs_layout_passes=False),
    )(tabT, tail_pad)


_SUB = 128
_CHUNK = 512


def _gather_body(idx_hbm, q_hbm, out_hbm, idx_v, rows_v, gsem, bpw):
    wid = lax.axis_index("s") * _NC + lax.axis_index("c")
    base = wid * bpw
    pltpu.sync_copy(idx_hbm.at[pl.ds(base, bpw)], idx_v)

    def chunk(ci, _):
        off = ci * _CHUNK
        handles = []
        for j in range(_CHUNK // _SUB):
            handles.append(pltpu.async_copy(
                q_hbm.at[idx_v.at[pl.ds(off + j * _SUB, _SUB)]],
                rows_v.at[pl.ds(j * _SUB, _SUB)],
                gsem,
            ))
        for h in handles:
            h.wait()
        pltpu.sync_copy(rows_v, out_hbm.at[pl.ds(base + off, _CHUNK)])
        return ()

    lax.fori_loop(0, bpw // _CHUNK, chunk, (), unroll=False)


def _sc_gather(idx_flat, q):
    n = idx_flat.shape[0]
    assert n % (_NW * _CHUNK) == 0
    bpw = n // _NW
    mesh = plsc.VectorSubcoreMesh(core_axis_name="c", subcore_axis_name="s")
    body = functools.partial(_gather_body, bpw=bpw)
    return pl.kernel(
        body,
        out_type=jax.ShapeDtypeStruct((n, 128), jnp.float32),
        mesh=mesh,
        scratch_types=[
            pltpu.VMEM((bpw,), jnp.int32),
            pltpu.VMEM((_CHUNK, 128), jnp.float32),
            pltpu.SemaphoreType.DMA,
        ],
    )(idx_flat, q)


_ROWS = 2048


def _proj_body(emb_ref, w_ref, b_ref, out_ref):
    e = emb_ref[0][:, 0:_DIM]
    acc = lax.dot_general(
        w_ref[...], e, (((0,), (1,)), ((), ())),
        preferred_element_type=jnp.float32,
    )
    acc = acc + b_ref[...]
    out_ref[0] = acc * jax.nn.sigmoid(acc)


def _tc_project(emb3, W, bcol):
    F, B = emb3.shape[0], emb3.shape[1]
    return pl.pallas_call(
        _proj_body,
        grid=(F, B // _ROWS),
        in_specs=[
            pl.BlockSpec((1, _ROWS, 128), lambda f, i: (f, i, 0)),
            pl.BlockSpec((_DIM, _DIM), lambda f, i: (0, 0)),
            pl.BlockSpec((_DIM, 1), lambda f, i: (0, 0)),
        ],
        out_specs=pl.BlockSpec((1, _DIM, _ROWS), lambda f, i: (f, 0, i)),
        out_shape=jax.ShapeDtypeStruct((F, _DIM, B), jnp.float32),
    )(emb3, W, bcol)


def kernel(x, table, W, b):
    B, F = x.shape
    idx_flat = x.T.reshape(-1)
    tabT = table.T
    tail_pad = jnp.pad(table[_NFULL * 128:, :], ((0, 0), (0, 128 - _DIM)))
    q = _sc_transpose(tabT, tail_pad)
    emb = _sc_gather(idx_flat, q)
    emb3 = emb.reshape(F, B, 128)
    out3 = _tc_project(emb3, W, b.reshape(_DIM, 1))
    return out3.transpose(2, 0, 1)


# speedup vs baseline: 1.2300x; 1.0017x over previous
import functools

import jax
import jax.numpy as jnp
from jax import lax
from jax.experimental import pallas as pl
from jax.experimental.pallas import tpu as pltpu
from jax.experimental.pallas import tpu_sc as plsc

_DIM = 64
_VOCAB = 1000000
_NFULL = _VOCAB // 128
_TAIL = _VOCAB - _NFULL * 128


_info = plsc.get_sparse_core_info()
_NC, _NS = _info.num_cores, _info.num_subcores
_NW = _NC * _NS


_PITCH = 129


def _transpose_cols(slab, stage, n):
    def row(v, _):
        for c in range(4):
            rows16 = lax.iota(jnp.int32, 16) + (c * 16)
            cols16 = lax.iota(jnp.int32, 16) * 0 + v
            vals = plsc.load_gather(slab, [rows16, cols16])
            stage[v, pl.ds(c * 16, 16)] = vals
        return ()
    lax.fori_loop(0, n, row, (), unroll=8)


def _tr_body(tabT_hbm, q_hbm, slabs, stages, isem, osem, tail_hbm):
    wid = lax.axis_index("s") * _NC + lax.axis_index("c")

    def in_copy(b, d):
        return pltpu.make_async_copy(
            tabT_hbm.at[:, pl.ds(b * 128, 128)],
            slabs[d].at[:, pl.ds(0, 128)], isem)

    def out_copy(b, d):
        return pltpu.make_async_copy(
            stages[d], q_hbm.at[pl.ds(b * 128, 128)], osem)

    in_copy(wid, 0).start()
    in_copy(wid + _NW, 1).start()

    nblk = _NFULL // _NW + 1

    def blk2(k0, _):
        for d in range(2):
            k = k0 + d
            b = wid + k * _NW

            @pl.when(b < _NFULL)
            def _():
                in_copy(b, d).wait()

                @pl.when(k >= 2)
                def _():
                    out_copy(b - 2 * _NW, d).wait()
                _transpose_cols(slabs[d], stages[d], 128)
                out_copy(b, d).start()

                @pl.when(b + 2 * _NW < _NFULL)
                def _():
                    in_copy(b + 2 * _NW, d).start()
        return ()

    lax.fori_loop(0, nblk // 2 + 1, lambda k, c: blk2(2 * k, c), (),
                  unroll=False)

    nv = _NFULL // _NW + jnp.where(wid < _NFULL % _NW, 1, 0)
    out_copy(wid + (nv - 2) * _NW, 0).wait()
    out_copy(wid + (nv - 1) * _NW, 1).wait()

    @pl.when(wid == 0)
    def _():
        pltpu.sync_copy(tail_hbm, stages[0].at[pl.ds(0, _TAIL)])
        pltpu.sync_copy(stages[0].at[pl.ds(0, _TAIL)],
                        q_hbm.at[pl.ds(_NFULL * 128, _TAIL)])


def _tr_body_wrap(tabT_hbm, tail_hbm, q_hbm, slab_a, slab_b, stage_a,
                  stage_b, isem, osem):
    _tr_body(tabT_hbm, q_hbm, (slab_a, slab_b), (stage_a, stage_b),
             isem, osem, tail_hbm)


def _sc_transpose(tabT, tail_pad):
    mesh = plsc.VectorSubcoreMesh(core_axis_name="c", subcore_axis_name="s")
    return pl.kernel(
        _tr_body_wrap,
        out_type=jax.ShapeDtypeStruct((_VOCAB, 128), jnp.float32),
        mesh=mesh,
        scratch_types=[
            pltpu.VMEM((_DIM, _PITCH), jnp.float32),
            pltpu.VMEM((_DIM, _PITCH), jnp.float32),
            pltpu.VMEM((128, 128), jnp.float32),
            pltpu.VMEM((128, 128), jnp.float32),
            pltpu.SemaphoreType.DMA,
            pltpu.SemaphoreType.DMA,
        ],
        compiler_params=pltpu.CompilerParams(needs_layout_passes=False),
    )(tabT, tail_pad)


_SUB = 128
_CHUNK = 512


def _gather_body(idx_hbm, q_hbm, out_hbm, idx_v, rows_v, gsem, bpw):
    wid = lax.axis_index("s") * _NC + lax.axis_index("c")
    base = wid * bpw
    pltpu.sync_copy(idx_hbm.at[pl.ds(base, bpw)], idx_v)

    def chunk(ci, _):
        off = ci * _CHUNK
        handles = []
        for j in range(_CHUNK // _SUB):
            handles.append(pltpu.async_copy(
                q_hbm.at[idx_v.at[pl.ds(off + j * _SUB, _SUB)]],
                rows_v.at[pl.ds(j * _SUB, _SUB)],
                gsem,
            ))
        for h in handles:
            h.wait()
        pltpu.sync_copy(rows_v, out_hbm.at[pl.ds(base + off, _CHUNK)])
        return ()

    lax.fori_loop(0, bpw // _CHUNK, chunk, (), unroll=False)


def _sc_gather(idx_flat, q):
    n = idx_flat.shape[0]
    assert n % (_NW * _CHUNK) == 0
    bpw = n // _NW
    mesh = plsc.VectorSubcoreMesh(core_axis_name="c", subcore_axis_name="s")
    body = functools.partial(_gather_body, bpw=bpw)
    return pl.kernel(
        body,
        out_type=jax.ShapeDtypeStruct((n, 128), jnp.float32),
        mesh=mesh,
        scratch_types=[
            pltpu.VMEM((bpw,), jnp.int32),
            pltpu.VMEM((_CHUNK, 128), jnp.float32),
            pltpu.SemaphoreType.DMA,
        ],
    )(idx_flat, q)


_ROWS = 2048


def _proj_body(emb_ref, w_ref, b_ref, out_ref):
    e = emb_ref[0][:, 0:_DIM]
    acc = lax.dot_general(
        w_ref[...], e, (((0,), (1,)), ((), ())),
        preferred_element_type=jnp.float32,
    )
    acc = acc + b_ref[...]
    out_ref[0] = acc * jax.nn.sigmoid(acc)


def _tc_project(emb3, W, bcol):
    F, B = emb3.shape[0], emb3.shape[1]
    return pl.pallas_call(
        _proj_body,
        grid=(F, B // _ROWS),
        in_specs=[
            pl.BlockSpec((1, _ROWS, 128), lambda f, i: (f, i, 0)),
            pl.BlockSpec((_DIM, _DIM), lambda f, i: (0, 0)),
            pl.BlockSpec((_DIM, 1), lambda f, i: (0, 0)),
        ],
        out_specs=pl.BlockSpec((1, _DIM, _ROWS), lambda f, i: (f, 0, i)),
        out_shape=jax.ShapeDtypeStruct((F, _DIM, B), jnp.float32),
    )(emb3, W, bcol)


def kernel(x, table, W, b):
    B, F = x.shape
    idx_flat = x.T.reshape(-1)
    tabT = table.T
    tail_pad = jnp.pad(table[_NFULL * 128:, :], ((0, 0), (0, 128 - _DIM)))
    q = _sc_transpose(tabT, tail_pad)
    emb = _sc_gather(idx_flat, q)
    emb3 = emb.reshape(F, B, 128)
    out3 = _tc_project(emb3, W, b.reshape(_DIM, 1))
    return out3.transpose(2, 0, 1)


# speedup vs baseline: 2.9144x; 2.3695x over previous
import functools

import jax
import jax.numpy as jnp
from jax import lax
from jax.experimental import pallas as pl
from jax.experimental.pallas import tpu as pltpu
from jax.experimental.pallas import tpu_sc as plsc

_DIM = 64
_VOCAB = 1000000
_CBLK = 7680
_HBLK = _CBLK // 2

_info = plsc.get_sparse_core_info()
_NC, _NS = _info.num_cores, _info.num_subcores
_NW = _NC * _NS


def _repack_body(tab_ref, eye_ref, out_ref):
    t = lax.dot_general(
        tab_ref[...], eye_ref[...], (((0,), (0,)), ((), ())),
        preferred_element_type=jnp.float32,
    )
    out_ref[:, 0:_DIM] = t[0:_HBLK]
    out_ref[:, _DIM:2 * _DIM] = t[_HBLK:_CBLK]


def _tc_repack(tabT, eye):
    return pl.pallas_call(
        _repack_body,
        grid=(pl.cdiv(_VOCAB, _CBLK),),
        in_specs=[
            pl.BlockSpec((_DIM, _CBLK), lambda g: (0, g)),
            pl.BlockSpec((_DIM, _DIM), lambda g: (0, 0)),
        ],
        out_specs=pl.BlockSpec((_HBLK, 2 * _DIM), lambda g: (g, 0)),
        out_shape=jax.ShapeDtypeStruct(
            (pl.cdiv(_VOCAB, _CBLK) * _HBLK, 2 * _DIM), jnp.float32),
    )(tabT, eye)


_CHUNK = 256
_SUB = 128


def _gather_body(idx_hbm, q_hbm, out_hbm, work_v, hoff_v, rows_a, rows_b,
                 emb_v, gsem, bpw):
    wid = lax.axis_index("s") * _NC + lax.axis_index("c")
    base = wid * bpw
    pltpu.sync_copy(idx_hbm.at[pl.ds(base, bpw)], work_v)

    def xform(i, _):
        v = work_v[pl.ds(i * 16, 16)]
        g = v // _CBLK
        r = v - g * _CBLK
        h = jnp.where(r >= _HBLK, 1, 0)
        work_v[pl.ds(i * 16, 16)] = g * _HBLK + r - h * _HBLK
        hoff_v[pl.ds(i * 16, 16)] = h * _DIM
        return ()
    lax.fori_loop(0, bpw // 16, xform, (), unroll=8)

    rows_bufs = (rows_a, rows_b)

    def fire(ci, d):
        off = ci * _CHUNK
        for j in range(_CHUNK // _SUB):
            pltpu.async_copy(
                q_hbm.at[work_v.at[pl.ds(off + j * _SUB, _SUB)]],
                rows_bufs[d].at[pl.ds(j * _SUB, _SUB)],
                gsem,
            )

    def drain(ci, d):
        off = ci * _CHUNK
        for j in range(_CHUNK // _SUB):
            pltpu.make_async_copy(
                q_hbm.at[work_v.at[pl.ds(off + j * _SUB, _SUB)]],
                rows_bufs[d].at[pl.ds(j * _SUB, _SUB)],
                gsem,
            ).wait()

    nchunk = bpw // _CHUNK
    fire(0, 0)

    def chunk2(c0, _):
        for d in range(2):
            ci = c0 + d

            @pl.when(ci < nchunk)
            def _():
                drain(ci, d)

                @pl.when(ci + 1 < nchunk)
                def _():
                    fire(ci + 1, 1 - d)

                off = ci * _CHUNK
                rows = rows_bufs[d]

                def extract16(r16, _):
                    r0 = r16 * 16
                    hv = hoff_v[pl.ds(off + r0, 16)]
                    for l in range(16):
                        hw = hv[l]
                        for c in range(4):
                            emb_v[r0 + l, pl.ds(c * 16, 16)] = (
                                rows[r0 + l, pl.ds(hw + c * 16, 16)])
                    return ()
                lax.fori_loop(0, _CHUNK // 16, extract16, (), unroll=False)
                pltpu.sync_copy(emb_v, out_hbm.at[pl.ds(base + off, _CHUNK)])
        return ()

    lax.fori_loop(0, nchunk // 2, lambda k, c: chunk2(2 * k, c), (),
                  unroll=False)


def _sc_gather(idx_flat, q):
    n = idx_flat.shape[0]
    assert n % (_NW * _CHUNK) == 0
    bpw = n // _NW
    mesh = plsc.VectorSubcoreMesh(core_axis_name="c", subcore_axis_name="s")
    body = functools.partial(_gather_body, bpw=bpw)
    return pl.kernel(
        body,
        out_type=jax.ShapeDtypeStruct((n, _DIM), jnp.float32),
        mesh=mesh,
        scratch_types=[
            pltpu.VMEM((bpw,), jnp.int32),
            pltpu.VMEM((bpw,), jnp.int32),
            pltpu.VMEM((_CHUNK, 2 * _DIM), jnp.float32),
            pltpu.VMEM((_CHUNK, 2 * _DIM), jnp.float32),
            pltpu.VMEM((_CHUNK, _DIM), jnp.float32),
            pltpu.SemaphoreType.DMA,
        ],
        compiler_params=pltpu.CompilerParams(needs_layout_passes=False),
    )(idx_flat, q)


_ROWS = 2048


def _proj_body(emb_ref, w_ref, b_ref, out_ref):
    e = emb_ref[0]
    acc = lax.dot_general(
        w_ref[...], e, (((0,), (1,)), ((), ())),
        preferred_element_type=jnp.float32,
    )
    acc = acc + b_ref[...]
    out_ref[0] = acc * jax.nn.sigmoid(acc)


def _tc_project(emb3, W, bcol):
    F, B = emb3.shape[0], emb3.shape[1]
    return pl.pallas_call(
        _proj_body,
        grid=(F, B // _ROWS),
        in_specs=[
            pl.BlockSpec((1, _ROWS, _DIM), lambda f, i: (f, i, 0)),
            pl.BlockSpec((_DIM, _DIM), lambda f, i: (0, 0)),
            pl.BlockSpec((_DIM, 1), lambda f, i: (0, 0)),
        ],
        out_specs=pl.BlockSpec((1, _DIM, _ROWS), lambda f, i: (f, 0, i)),
        out_shape=jax.ShapeDtypeStruct((F, _DIM, B), jnp.float32),
    )(emb3, W, bcol)


def kernel(x, table, W, b):
    B, F = x.shape
    idx_flat = x.T.reshape(-1)
    tabT = table.T
    eye = jnp.eye(_DIM, dtype=jnp.float32)
    q = _tc_repack(tabT, eye)
    emb = _sc_gather(idx_flat, q)
    emb3 = emb.reshape(F, B, _DIM)
    out3 = _tc_project(emb3, W, b.reshape(_DIM, 1))
    return out3.transpose(2, 0, 1)


# speedup vs baseline: 3.4665x; 1.1894x over previous
import functools

import jax
import jax.numpy as jnp
from jax import lax
from jax.experimental import pallas as pl
from jax.experimental.pallas import tpu as pltpu
from jax.experimental.pallas import tpu_sc as plsc

_DIM = 64
_VOCAB = 1000000
_CBLK = 15360
_HBLK = _CBLK // 2

_info = plsc.get_sparse_core_info()
_NC, _NS = _info.num_cores, _info.num_subcores
_NW = _NC * _NS


def _repack_body(tab_ref, eye_ref, out_ref):
    t = lax.dot_general(
        tab_ref[...], eye_ref[...], (((0,), (0,)), ((), ())),
        preferred_element_type=jnp.float32,
    )
    out_ref[:, 0:_DIM] = t[0:_HBLK]
    out_ref[:, _DIM:2 * _DIM] = t[_HBLK:_CBLK]


def _tc_repack(tabT, eye):
    return pl.pallas_call(
        _repack_body,
        grid=(pl.cdiv(_VOCAB, _CBLK),),
        in_specs=[
            pl.BlockSpec((_DIM, _CBLK), lambda g: (0, g)),
            pl.BlockSpec((_DIM, _DIM), lambda g: (0, 0)),
        ],
        out_specs=pl.BlockSpec((_HBLK, 2 * _DIM), lambda g: (g, 0)),
        out_shape=jax.ShapeDtypeStruct(
            (pl.cdiv(_VOCAB, _CBLK) * _HBLK, 2 * _DIM), jnp.float32),
    )(tabT, eye)


_CHUNK = 256
_SUB = 128


def _gather_body(idx_hbm, q_hbm, out_hbm, work_v, hoff_v, rows_a, rows_b,
                 emb_v, gsem, bpw):
    wid = lax.axis_index("s") * _NC + lax.axis_index("c")
    base = wid * bpw
    pltpu.sync_copy(idx_hbm.at[pl.ds(base, bpw)], work_v)

    def xform(i, _):
        v = work_v[pl.ds(i * 16, 16)]
        g = v // _CBLK
        r = v - g * _CBLK
        h = jnp.where(r >= _HBLK, 1, 0)
        work_v[pl.ds(i * 16, 16)] = g * _HBLK + r - h * _HBLK
        hoff_v[pl.ds(i * 16, 16)] = h * _DIM
        return ()
    lax.fori_loop(0, bpw // 16, xform, (), unroll=8)

    rows_bufs = (rows_a, rows_b)

    def fire(ci, d):
        off = ci * _CHUNK
        for j in range(_CHUNK // _SUB):
            pltpu.async_copy(
                q_hbm.at[work_v.at[pl.ds(off + j * _SUB, _SUB)]],
                rows_bufs[d].at[pl.ds(j * _SUB, _SUB)],
                gsem,
            )

    def drain(ci, d):
        off = ci * _CHUNK
        for j in range(_CHUNK // _SUB):
            pltpu.make_async_copy(
                q_hbm.at[work_v.at[pl.ds(off + j * _SUB, _SUB)]],
                rows_bufs[d].at[pl.ds(j * _SUB, _SUB)],
                gsem,
            ).wait()

    nchunk = bpw // _CHUNK
    fire(0, 0)

    def chunk2(c0, _):
        for d in range(2):
            ci = c0 + d

            @pl.when(ci < nchunk)
            def _():
                drain(ci, d)

                @pl.when(ci + 1 < nchunk)
                def _():
                    fire(ci + 1, 1 - d)

                off = ci * _CHUNK
                rows = rows_bufs[d]

                def extract16(r16, _):
                    r0 = r16 * 16
                    hv = hoff_v[pl.ds(off + r0, 16)]
                    for l in range(16):
                        hw = hv[l]
                        for c in range(4):
                            emb_v[r0 + l, pl.ds(c * 16, 16)] = (
                                rows[r0 + l, pl.ds(hw + c * 16, 16)])
                    return ()
                lax.fori_loop(0, _CHUNK // 16, extract16, (), unroll=False)
                pltpu.sync_copy(emb_v, out_hbm.at[pl.ds(base + off, _CHUNK)])
        return ()

    lax.fori_loop(0, nchunk // 2, lambda k, c: chunk2(2 * k, c), (),
                  unroll=False)


def _sc_gather(idx_flat, q):
    n = idx_flat.shape[0]
    assert n % (_NW * _CHUNK) == 0
    bpw = n // _NW
    mesh = plsc.VectorSubcoreMesh(core_axis_name="c", subcore_axis_name="s")
    body = functools.partial(_gather_body, bpw=bpw)
    return pl.kernel(
        body,
        out_type=jax.ShapeDtypeStruct((n, _DIM), jnp.float32),
        mesh=mesh,
        scratch_types=[
            pltpu.VMEM((bpw,), jnp.int32),
            pltpu.VMEM((bpw,), jnp.int32),
            pltpu.VMEM((_CHUNK, 2 * _DIM), jnp.float32),
            pltpu.VMEM((_CHUNK, 2 * _DIM), jnp.float32),
            pltpu.VMEM((_CHUNK, _DIM), jnp.float32),
            pltpu.SemaphoreType.DMA,
        ],
        compiler_params=pltpu.CompilerParams(needs_layout_passes=False),
    )(idx_flat, q)


_ROWS = 8192


def _proj_body(emb_ref, w_ref, b_ref, out_ref):
    e = emb_ref[0]
    acc = lax.dot_general(
        w_ref[...], e, (((0,), (1,)), ((), ())),
        preferred_element_type=jnp.float32,
    )
    acc = acc + b_ref[...]
    out_ref[0] = acc * jax.nn.sigmoid(acc)


def _tc_project(emb3, W, bcol):
    F, B = emb3.shape[0], emb3.shape[1]
    return pl.pallas_call(
        _proj_body,
        grid=(F, B // _ROWS),
        in_specs=[
            pl.BlockSpec((1, _ROWS, _DIM), lambda f, i: (f, i, 0)),
            pl.BlockSpec((_DIM, _DIM), lambda f, i: (0, 0)),
            pl.BlockSpec((_DIM, 1), lambda f, i: (0, 0)),
        ],
        out_specs=pl.BlockSpec((1, _DIM, _ROWS), lambda f, i: (f, 0, i)),
        out_shape=jax.ShapeDtypeStruct((F, _DIM, B), jnp.float32),
    )(emb3, W, bcol)


def kernel(x, table, W, b):
    B, F = x.shape
    idx_flat = x.T.reshape(-1)
    tabT = table.T
    eye = jnp.eye(_DIM, dtype=jnp.float32)
    q = _tc_repack(tabT, eye)
    emb = _sc_gather(idx_flat, q)
    emb3 = emb.reshape(F, B, _DIM)
    out3 = _tc_project(emb3, W, b.reshape(_DIM, 1))
    return out3.transpose(2, 0, 1)
